# baseline (speedup 1.0000x reference)
import functools

import jax
import jax.numpy as jnp
from jax import lax
from jax.experimental import pallas as pl
from jax.experimental.pallas import tpu as pltpu
from jax.experimental.pallas import tpu_sc as plsc

VOCAB = 100000
EMBED = 64
BATCH = 16384

NUM_CORES = 2
NUM_SUBCORES = 16
LANES = 16
NUM_WORKERS = NUM_CORES * NUM_SUBCORES
B_PER_W = BATCH // NUM_WORKERS
GATHER_CHUNK = 128
NUM_CHUNKS = B_PER_W // GATHER_CHUNK
GROUPS = B_PER_W // LANES


def _body(target_hbm, context_hbm, table_hbm, out_hbm,
          tgt_idx, ctx_idx, t_rows, c_rows, out_v, sem):
    wid = lax.axis_index("s") * NUM_CORES + lax.axis_index("c")
    base = wid * B_PER_W

    pltpu.sync_copy(target_hbm.at[pl.ds(base, B_PER_W)], tgt_idx)
    pltpu.sync_copy(context_hbm.at[pl.ds(base, B_PER_W)], ctx_idx)

    copies = []
    for ch in range(NUM_CHUNKS):
        sl = pl.ds(ch * GATHER_CHUNK, GATHER_CHUNK)
        copies.append(pltpu.async_copy(
            table_hbm.at[tgt_idx.at[sl]], t_rows.at[sl], sem))
        copies.append(pltpu.async_copy(
            table_hbm.at[ctx_idx.at[sl]], c_rows.at[sl], sem))
    for cp in copies:
        cp.wait()

    lane = lax.iota(jnp.int32, LANES)

    def group(g, _):
        scores = jnp.zeros((LANES,), jnp.float32)
        for r in range(LANES):
            b = g * LANES + r
            s = jnp.zeros((LANES,), jnp.float32)
            for k in range(EMBED // LANES):
                tv = t_rows[b, pl.ds(k * LANES, LANES)]
                cv = c_rows[b, pl.ds(k * LANES, LANES)]
                s = s + tv * cv
            tot = jnp.sum(s)
            scores = jnp.where(lane == r, tot, scores)
        out_v[pl.ds(g * LANES, LANES)] = scores
        return _

    lax.fori_loop(0, GROUPS, group, 0, unroll=False)

    pltpu.sync_copy(out_v, out_hbm.at[pl.ds(base, B_PER_W)])


@jax.jit
def kernel(target, context, word_embeddings):
    mesh = plsc.VectorSubcoreMesh(core_axis_name="c", subcore_axis_name="s")
    run = pl.kernel(
        _body,
        out_type=jax.ShapeDtypeStruct((BATCH,), jnp.float32),
        mesh=mesh,
        scratch_types=[
            pltpu.VMEM((B_PER_W,), jnp.int32),
            pltpu.VMEM((B_PER_W,), jnp.int32),
            pltpu.VMEM((B_PER_W, EMBED), jnp.float32),
            pltpu.VMEM((B_PER_W, EMBED), jnp.float32),
            pltpu.VMEM((B_PER_W,), jnp.float32),
            pltpu.SemaphoreType.DMA,
        ],
        compiler_params=pltpu.CompilerParams(
            needs_layout_passes=False, use_tc_tiling_on_sc=False),
    )
    return run(target, context, word_embeddings)


# speedup vs baseline: 1.3210x; 1.3210x over previous
import jax
import jax.numpy as jnp
from jax import lax
from jax.experimental import pallas as pl
from jax.experimental.pallas import tpu as pltpu
from jax.experimental.pallas import tpu_sc as plsc

VOCAB = 100000
EMBED = 64
BATCH = 16384

NUM_CORES = 2
NUM_SUBCORES = 16
LANES = 16
NUM_WORKERS = NUM_CORES * NUM_SUBCORES
B_PER_W = BATCH // NUM_WORKERS
BB = 16
PASSES = 2
B_PER_PASS = B_PER_W // PASSES
NB = B_PER_PASS // BB


def _body(target_hbm, context_hbm, table_hbm, out_hbm,
          tgt_idx, ctx_idx, t_rows, c_rows, out_v,
          sem_a, sem_b):
    wid = lax.axis_index("s") * NUM_CORES + lax.axis_index("c")
    base = wid * B_PER_W

    pltpu.sync_copy(target_hbm.at[pl.ds(base, B_PER_W)], tgt_idx)
    pltpu.sync_copy(context_hbm.at[pl.ds(base, B_PER_W)], ctx_idx)

    lane = lax.iota(jnp.int32, LANES)

    for p in range(PASSES):
        pbase = p * B_PER_PASS

        def fire(g, sem):
            iv_t = tgt_idx[pl.ds(pbase + g * BB, BB)]
            iv_c = ctx_idx[pl.ds(pbase + g * BB, BB)]
            for j in range(BB):
                pltpu.async_copy(
                    table_hbm.at[iv_t[j]], t_rows.at[g * BB + j], sem)
                pltpu.async_copy(
                    table_hbm.at[iv_c[j]], c_rows.at[g * BB + j], sem)

        def drain(g, sem):
            sl = pl.ds(g * BB, BB)
            pltpu.make_async_copy(
                table_hbm.at[pl.ds(0, BB)], t_rows.at[sl], sem).wait()
            pltpu.make_async_copy(
                table_hbm.at[pl.ds(0, BB)], c_rows.at[sl], sem).wait()

        def compute(g):
            scores = jnp.zeros((LANES,), jnp.float32)
            for r in range(LANES):
                b = g * LANES + r
                s = jnp.zeros((LANES,), jnp.float32)
                for k in range(EMBED // LANES):
                    tv = t_rows[b, pl.ds(k * LANES, LANES)]
                    cv = c_rows[b, pl.ds(k * LANES, LANES)]
                    s = s + tv * cv
                tot = jnp.sum(s)
                scores = jnp.where(lane == r, tot, scores)
            out_v[pl.ds(pbase + g * LANES, LANES)] = scores

        fire(0, sem_a)
        fire(1, sem_b)

        def step(k, carry):
            g0 = 2 * k

            @pl.when(g0 + 2 < NB)
            def _():
                fire(g0 + 2, sem_a)
            drain(g0, sem_a)
            compute(g0)

            g1 = g0 + 1

            @pl.when(g1 + 2 < NB)
            def _():
                fire(g1 + 2, sem_b)
            drain(g1, sem_b)
            compute(g1)
            return carry

        lax.fori_loop(0, NB // 2, step, 0, unroll=False)

    pltpu.sync_copy(out_v, out_hbm.at[pl.ds(base, B_PER_W)])


@jax.jit
def kernel(target, context, word_embeddings):
    mesh = plsc.VectorSubcoreMesh(core_axis_name="c", subcore_axis_name="s")
    run = pl.kernel(
        _body,
        out_type=jax.ShapeDtypeStruct((BATCH,), jnp.float32),
        mesh=mesh,
        scratch_types=[
            pltpu.VMEM((B_PER_W,), jnp.int32),
            pltpu.VMEM((B_PER_W,), jnp.int32),
            pltpu.VMEM((B_PER_PASS, EMBED), jnp.float32),
            pltpu.VMEM((B_PER_PASS, EMBED), jnp.float32),
            pltpu.VMEM((B_PER_W,), jnp.float32),
            pltpu.SemaphoreType.DMA,
            pltpu.SemaphoreType.DMA,
        ],
        compiler_params=pltpu.CompilerParams(
            needs_layout_passes=False, use_tc_tiling_on_sc=True),
    )
    return run(target, context, word_embeddings)


# speedup vs baseline: 1.3451x; 1.0183x over previous
import jax
import jax.numpy as jnp
from jax import lax
from jax.experimental import pallas as pl
from jax.experimental.pallas import tpu as pltpu
from jax.experimental.pallas import tpu_sc as plsc

VOCAB = 100000
EMBED = 64
BATCH = 16384

NUM_CORES = 2
NUM_SUBCORES = 16
LANES = 16
B_PER_CORE = BATCH // NUM_CORES
D_PER_SUB = EMBED // NUM_SUBCORES
GROUPS = B_PER_CORE // LANES
ACC_ROWS = B_PER_CORE // 128
STRIPE_ROWS = ACC_ROWS // NUM_SUBCORES


def _body(target_hbm, context_hbm, wt_hbm, out_hbm,
          t_idx, c_idx, rowbuf, acc, idxbuf, shared, sem):
    c = lax.axis_index("c")
    s = lax.axis_index("s")
    base = c * B_PER_CORE

    pltpu.sync_copy(target_hbm.at[pl.ds(base, B_PER_CORE)], t_idx)
    pltpu.sync_copy(context_hbm.at[pl.ds(base, B_PER_CORE)], c_idx)

    for k in range(ACC_ROWS // LANES):
        idxbuf[pl.ds(k * LANES, LANES)] = (
            lax.iota(jnp.int32, LANES) + k * LANES)

    for dd in range(D_PER_SUB):
        d = s * D_PER_SUB + dd
        pltpu.sync_copy(wt_hbm.at[d], rowbuf)

        def group(g, carry, first=(dd == 0)):
            sl = pl.ds(g * LANES, LANES)
            tv = t_idx[sl]
            cv = c_idx[sl]
            tg = plsc.load_gather(rowbuf, [tv])
            cg = plsc.load_gather(rowbuf, [cv])
            prod = tg * cg
            row = g // 8
            col = (g % 8) * LANES
            if first:
                acc[row, pl.ds(col, LANES)] = prod
            else:
                plsc.addupdate(acc.at[row, pl.ds(col, LANES)], prod)
            return carry

        lax.fori_loop(0, GROUPS, group, 0, unroll=False)

    @pl.when(s == 0)
    def _():
        pltpu.sync_copy(acc, shared)

    plsc.subcore_barrier()

    @pl.when(s != 0)
    def _():
        cp = pltpu.make_async_copy(acc, shared.at[idxbuf], sem)
        cp.start(add=True)
        cp.wait()

    plsc.subcore_barrier()

    rsl = pl.ds(s * STRIPE_ROWS, STRIPE_ROWS)
    osl = pl.ds(c * ACC_ROWS + s * STRIPE_ROWS, STRIPE_ROWS)
    pltpu.sync_copy(shared.at[rsl], out_hbm.at[osl])


@jax.jit
def kernel(target, context, word_embeddings):
    wt = word_embeddings.T
    mesh = plsc.VectorSubcoreMesh(core_axis_name="c", subcore_axis_name="s")
    run = pl.kernel(
        _body,
        out_type=jax.ShapeDtypeStruct((BATCH // 128, 128), jnp.float32),
        mesh=mesh,
        scratch_types=[
            pltpu.VMEM((B_PER_CORE,), jnp.int32),
            pltpu.VMEM((B_PER_CORE,), jnp.int32),
            pltpu.VMEM((VOCAB,), jnp.float32),
            pltpu.VMEM((ACC_ROWS, 128), jnp.float32),
            pltpu.VMEM((ACC_ROWS,), jnp.int32),
            pltpu.VMEM_SHARED((ACC_ROWS, 128), jnp.float32),
            pltpu.SemaphoreType.DMA,
        ],
        compiler_params=pltpu.CompilerParams(
            needs_layout_passes=False, use_tc_tiling_on_sc=True),
    )
    out = run(target, context, wt)
    return jnp.reshape(out, (BATCH,))


# speedup vs baseline: 1.4945x; 1.1110x over previous
import jax
import jax.numpy as jnp
from jax import lax
from jax.experimental import pallas as pl
from jax.experimental.pallas import tpu as pltpu
from jax.experimental.pallas import tpu_sc as plsc

VOCAB = 100000
EMBED = 64
BATCH = 16384

NUM_CORES = 2
NUM_SUBCORES = 16
LANES = 16
B_PER_CORE = BATCH // NUM_CORES
D_PER_SUB = EMBED // NUM_SUBCORES
GROUPS = B_PER_CORE // LANES
ACC_ROWS = B_PER_CORE // 128
STRIPE_ROWS = ACC_ROWS // NUM_SUBCORES


def _body(target_hbm, context_hbm, wt_hbm, out_hbm,
          t_idx, c_idx, rowbuf, acc, idxbuf, shared, sem):
    c = lax.axis_index("c")
    s = lax.axis_index("s")
    base = c * B_PER_CORE

    pltpu.sync_copy(target_hbm.at[pl.ds(base, B_PER_CORE)], t_idx)
    pltpu.sync_copy(context_hbm.at[pl.ds(base, B_PER_CORE)], c_idx)

    for k in range(ACC_ROWS // LANES):
        idxbuf[pl.ds(k * LANES, LANES)] = (
            lax.iota(jnp.int32, LANES) + k * LANES)

    for dd in range(D_PER_SUB):
        d = s * D_PER_SUB + dd
        pltpu.sync_copy(wt_hbm.at[d], rowbuf)

        def group(k, carry, first=(dd == 0)):
            for j in range(8):
                sl = pl.ds((k * 8 + j) * LANES, LANES)
                tv = t_idx[sl]
                cv = c_idx[sl]
                tg = plsc.load_gather(rowbuf, [tv])
                cg = plsc.load_gather(rowbuf, [cv])
                prod = tg * cg
                if first:
                    acc[k, pl.ds(j * LANES, LANES)] = prod
                else:
                    plsc.addupdate(acc.at[k, pl.ds(j * LANES, LANES)], prod)
            return carry

        lax.fori_loop(0, GROUPS // 8, group, 0, unroll=False)

    @pl.when(s == 0)
    def _():
        pltpu.sync_copy(acc, shared)

    plsc.subcore_barrier()

    @pl.when(s != 0)
    def _():
        cp = pltpu.make_async_copy(acc, shared.at[idxbuf], sem)
        cp.start(add=True)
        cp.wait()

    plsc.subcore_barrier()

    rsl = pl.ds(s * STRIPE_ROWS, STRIPE_ROWS)
    osl = pl.ds(c * ACC_ROWS + s * STRIPE_ROWS, STRIPE_ROWS)
    pltpu.sync_copy(shared.at[rsl], out_hbm.at[osl])


@jax.jit
def kernel(target, context, word_embeddings):
    wt = word_embeddings.T
    mesh = plsc.VectorSubcoreMesh(core_axis_name="c", subcore_axis_name="s")
    run = pl.kernel(
        _body,
        out_type=jax.ShapeDtypeStruct((BATCH // 128, 128), jnp.float32),
        mesh=mesh,
        scratch_types=[
            pltpu.VMEM((B_PER_CORE,), jnp.int32),
            pltpu.VMEM((B_PER_CORE,), jnp.int32),
            pltpu.VMEM((VOCAB,), jnp.float32),
            pltpu.VMEM((ACC_ROWS, 128), jnp.float32),
            pltpu.VMEM((ACC_ROWS,), jnp.int32),
            pltpu.VMEM_SHARED((ACC_ROWS, 128), jnp.float32),
            pltpu.SemaphoreType.DMA,
        ],
        compiler_params=pltpu.CompilerParams(
            needs_layout_passes=False, use_tc_tiling_on_sc=True),
    )
    out = run(target, context, wt)
    return jnp.reshape(out, (BATCH,))


# speedup vs baseline: 1.6751x; 1.1209x over previous
import jax
import jax.numpy as jnp
from jax import lax
from jax.experimental import pallas as pl
from jax.experimental.pallas import tpu as pltpu
from jax.experimental.pallas import tpu_sc as plsc

VOCAB = 100000
EMBED = 64
BATCH = 16384

NUM_CORES = 2
NUM_SUBCORES = 16
LANES = 16
B_PER_CORE = BATCH // NUM_CORES
D_PER_SUB = EMBED // NUM_SUBCORES
GROUPS = B_PER_CORE // LANES
ACC_ROWS = B_PER_CORE // 128
STRIPE_ROWS = ACC_ROWS // NUM_SUBCORES


def _body(target_hbm, context_hbm, wt_hbm, out_hbm,
          t_idx, c_idx, rowbuf, acc, idxbuf, shared, sem):
    c = lax.axis_index("c")
    s = lax.axis_index("s")
    base = c * B_PER_CORE

    pltpu.sync_copy(target_hbm.at[pl.ds(base, B_PER_CORE)], t_idx)
    pltpu.sync_copy(context_hbm.at[pl.ds(base, B_PER_CORE)], c_idx)

    for k in range(ACC_ROWS // LANES):
        idxbuf[pl.ds(k * LANES, LANES)] = (
            lax.iota(jnp.int32, LANES) + k * LANES)

    for dd in range(D_PER_SUB):
        d = s * D_PER_SUB + dd
        pltpu.sync_copy(wt_hbm.at[d], rowbuf)

        first = dd == 0

        @plsc.parallel_loop(0, GROUPS // 8, unroll=2)
        def _(k):
            for j in range(8):
                sl = pl.ds((k * 8 + j) * LANES, LANES)
                tv = t_idx[sl]
                cv = c_idx[sl]
                tg = plsc.load_gather(rowbuf, [tv])
                cg = plsc.load_gather(rowbuf, [cv])
                prod = tg * cg
                if first:
                    acc[k, pl.ds(j * LANES, LANES)] = prod
                else:
                    plsc.addupdate(acc.at[k, pl.ds(j * LANES, LANES)], prod)

    @pl.when(s == 0)
    def _():
        pltpu.sync_copy(acc, shared)

    plsc.subcore_barrier()

    @pl.when(s != 0)
    def _():
        cp = pltpu.make_async_copy(acc, shared.at[idxbuf], sem)
        cp.start(add=True)
        cp.wait()

    plsc.subcore_barrier()

    rsl = pl.ds(s * STRIPE_ROWS, STRIPE_ROWS)
    osl = pl.ds(c * ACC_ROWS + s * STRIPE_ROWS, STRIPE_ROWS)
    pltpu.sync_copy(shared.at[rsl], out_hbm.at[osl])


@jax.jit
def kernel(target, context, word_embeddings):
    wt = word_embeddings.T
    mesh = plsc.VectorSubcoreMesh(core_axis_name="c", subcore_axis_name="s")
    run = pl.kernel(
        _body,
        out_type=jax.ShapeDtypeStruct((BATCH // 128, 128), jnp.float32),
        mesh=mesh,
        scratch_types=[
            pltpu.VMEM((B_PER_CORE,), jnp.int32),
            pltpu.VMEM((B_PER_CORE,), jnp.int32),
            pltpu.VMEM((VOCAB,), jnp.float32),
            pltpu.VMEM((ACC_ROWS, 128), jnp.float32),
            pltpu.VMEM((ACC_ROWS,), jnp.int32),
            pltpu.VMEM_SHARED((ACC_ROWS, 128), jnp.float32),
            pltpu.SemaphoreType.DMA,
        ],
        compiler_params=pltpu.CompilerParams(
            needs_layout_passes=False, use_tc_tiling_on_sc=True),
    )
    out = run(target, context, wt)
    return jnp.reshape(out, (BATCH,))
